# baseline (speedup 1.0000x reference)
import functools

import jax
import jax.numpy as jnp
from jax import lax
from jax.experimental import pallas as pl
from jax.experimental.pallas import tpu as pltpu
from jax.experimental.pallas import tpu_sc as plsc

_E = 64
_TB = 512
_NC = 2
_NS = 16
_NW = _NC * _NS
_L = 16


def _matmul_body(x_ref, wt_ref, lg_ref):
    lg_ref[...] = jnp.dot(x_ref[...], wt_ref[...],
                          preferred_element_type=jnp.float32).T


def _gate_logits_t(x, WT):
    n, d = x.shape
    return pl.pallas_call(
        _matmul_body,
        grid=(n // _TB,),
        in_specs=[
            pl.BlockSpec((_TB, d), lambda i: (i, 0)),
            pl.BlockSpec((d, _E), lambda i: (0, 0)),
        ],
        out_specs=pl.BlockSpec((_E, _TB), lambda i: (0, i)),
        out_shape=jax.ShapeDtypeStruct((_E, n), jnp.float32),
        compiler_params=pltpu.CompilerParams(
            dimension_semantics=("arbitrary",),
        ),
    )(x, WT)


def _sc_top2_body(tpw, lg_hbm, i1_hbm, i2_hbm, w0_hbm, w1_hbm,
                  lg_v, i1_v, i2_v, w0_v, w1_v):
    wid = lax.axis_index("s") * _NC + lax.axis_index("c")
    base = wid * tpw
    pltpu.sync_copy(lg_hbm.at[:, pl.ds(base, tpw)], lg_v)

    def group(g, carry):
        tok = g * _L
        neg = jnp.full((_L,), -jnp.inf, jnp.float32)
        zero = jnp.zeros((_L,), jnp.int32)
        bestv, secondv = neg, neg
        besti, secondi = zero, zero
        for e in range(_E):
            col = jnp.full((_L,), e, jnp.int32)
            v = lg_v[e, pl.ds(tok, _L)]
            gt1 = v > bestv
            gt2 = v > secondv
            sv = jnp.where(gt2, v, secondv)
            si = jnp.where(gt2, col, secondi)
            secondv = jnp.where(gt1, bestv, sv)
            secondi = jnp.where(gt1, besti, si)
            bestv = jnp.where(gt1, v, bestv)
            besti = jnp.where(gt1, col, besti)
        ex = jnp.exp(secondv - bestv)
        w0 = 1.0 / (1.0 + ex)
        i1_v[pl.ds(tok, _L)] = besti
        i2_v[pl.ds(tok, _L)] = secondi
        w0_v[pl.ds(tok, _L)] = w0
        w1_v[pl.ds(tok, _L)] = 1.0 - w0
        return carry

    lax.fori_loop(0, tpw // _L, group, 0)
    pltpu.sync_copy(i1_v, i1_hbm.at[pl.ds(base, tpw)])
    pltpu.sync_copy(i2_v, i2_hbm.at[pl.ds(base, tpw)])
    pltpu.sync_copy(w0_v, w0_hbm.at[pl.ds(base, tpw)])
    pltpu.sync_copy(w1_v, w1_hbm.at[pl.ds(base, tpw)])


def _sc_top2(logits_t):
    n = logits_t.shape[1]
    tpw = n // _NW
    mesh = plsc.VectorSubcoreMesh(core_axis_name="c", subcore_axis_name="s")
    out_i = jax.ShapeDtypeStruct((n,), jnp.int32)
    out_f = jax.ShapeDtypeStruct((n,), jnp.float32)
    fn = pl.kernel(
        functools.partial(_sc_top2_body, tpw),
        out_type=[out_i, out_i, out_f, out_f],
        mesh=mesh,
        scratch_types=[
            pltpu.VMEM((_E, tpw), jnp.float32),
            pltpu.VMEM((tpw,), jnp.int32),
            pltpu.VMEM((tpw,), jnp.int32),
            pltpu.VMEM((tpw,), jnp.float32),
            pltpu.VMEM((tpw,), jnp.float32),
        ],
    )
    return fn(logits_t)


def kernel(x, W):
    logits_t = _gate_logits_t(x, W.T)
    i1, i2, w0, w1 = _sc_top2(logits_t)
    topi = jnp.stack([i1, i2], axis=1)
    weights = jnp.stack([w0, w1], axis=1)
    return (topi, weights)


# speedup vs baseline: 1.1307x; 1.1307x over previous
import functools

import jax
import jax.numpy as jnp
from jax import lax
from jax.experimental import pallas as pl
from jax.experimental.pallas import tpu as pltpu
from jax.experimental.pallas import tpu_sc as plsc

_E = 64
_TB = 1024
_NC = 2
_NS = 16
_NW = _NC * _NS
_L = 16


def _matmul_body(x_ref, wt_ref, lg_ref):
    lg_ref[...] = jnp.dot(x_ref[...], wt_ref[...],
                          preferred_element_type=jnp.float32).T


def _gate_logits_t(x, WT):
    n, d = x.shape
    return pl.pallas_call(
        _matmul_body,
        grid=(n // _TB,),
        in_specs=[
            pl.BlockSpec((_TB, d), lambda i: (i, 0)),
            pl.BlockSpec((d, _E), lambda i: (0, 0)),
        ],
        out_specs=pl.BlockSpec((_E, _TB), lambda i: (0, i)),
        out_shape=jax.ShapeDtypeStruct((_E, n), jnp.float32),
        compiler_params=pltpu.CompilerParams(
            dimension_semantics=("arbitrary",),
        ),
    )(x, WT)


def _sc_top2_body(tpw, lg_hbm, i1_hbm, i2_hbm, w0_hbm, w1_hbm,
                  lg_v, i1_v, i2_v, w0_v, w1_v):
    wid = lax.axis_index("s") * _NC + lax.axis_index("c")
    base = wid * tpw
    pltpu.sync_copy(lg_hbm.at[:, pl.ds(base, tpw)], lg_v)

    @plsc.parallel_loop(0, tpw // _L, unroll=4)
    def group(g):
        tok = g * _L
        neg = jnp.full((_L,), -jnp.inf, jnp.float32)
        zero = jnp.zeros((_L,), jnp.int32)
        bestv, secondv = neg, neg
        besti, secondi = zero, zero
        for e in range(_E):
            col = jnp.full((_L,), e, jnp.int32)
            v = lg_v[e, pl.ds(tok, _L)]
            gt1 = v > bestv
            gt2 = v > secondv
            sv = jnp.maximum(secondv, v)
            si = jnp.where(gt2, col, secondi)
            secondv = jnp.where(gt1, bestv, sv)
            secondi = jnp.where(gt1, besti, si)
            bestv = jnp.maximum(bestv, v)
            besti = jnp.where(gt1, col, besti)
        ex = jnp.exp(secondv - bestv)
        w0 = 1.0 / (1.0 + ex)
        i1_v[pl.ds(tok, _L)] = besti
        i2_v[pl.ds(tok, _L)] = secondi
        w0_v[pl.ds(tok, _L)] = w0
        w1_v[pl.ds(tok, _L)] = 1.0 - w0
    pltpu.sync_copy(i1_v, i1_hbm.at[pl.ds(base, tpw)])
    pltpu.sync_copy(i2_v, i2_hbm.at[pl.ds(base, tpw)])
    pltpu.sync_copy(w0_v, w0_hbm.at[pl.ds(base, tpw)])
    pltpu.sync_copy(w1_v, w1_hbm.at[pl.ds(base, tpw)])


def _sc_top2(logits_t):
    n = logits_t.shape[1]
    tpw = n // _NW
    mesh = plsc.VectorSubcoreMesh(core_axis_name="c", subcore_axis_name="s")
    out_i = jax.ShapeDtypeStruct((n,), jnp.int32)
    out_f = jax.ShapeDtypeStruct((n,), jnp.float32)
    fn = pl.kernel(
        functools.partial(_sc_top2_body, tpw),
        out_type=[out_i, out_i, out_f, out_f],
        mesh=mesh,
        scratch_types=[
            pltpu.VMEM((_E, tpw), jnp.float32),
            pltpu.VMEM((tpw,), jnp.int32),
            pltpu.VMEM((tpw,), jnp.int32),
            pltpu.VMEM((tpw,), jnp.float32),
            pltpu.VMEM((tpw,), jnp.float32),
        ],
    )
    return fn(logits_t)


def kernel(x, W):
    logits_t = _gate_logits_t(x, W.T)
    i1, i2, w0, w1 = _sc_top2(logits_t)
    topi = jnp.stack([i1, i2], axis=1)
    weights = jnp.stack([w0, w1], axis=1)
    return (topi, weights)


# speedup vs baseline: 1.1673x; 1.0323x over previous
import functools

import jax
import jax.numpy as jnp
from jax import lax
from jax.experimental import pallas as pl
from jax.experimental.pallas import tpu as pltpu
from jax.experimental.pallas import tpu_sc as plsc

_E = 64
_TB = 1024
_NC = 2
_NS = 16
_NW = _NC * _NS
_L = 16


def _matmul_body(x_ref, w_ref, lg_ref):
    lg_ref[...] = jax.lax.dot_general(
        w_ref[...], x_ref[...], (((1,), (1,)), ((), ())),
        preferred_element_type=jnp.float32)


def _gate_logits_t(x, W):
    n, d = x.shape
    return pl.pallas_call(
        _matmul_body,
        grid=(n // _TB,),
        in_specs=[
            pl.BlockSpec((_TB, d), lambda i: (i, 0)),
            pl.BlockSpec((_E, d), lambda i: (0, 0)),
        ],
        out_specs=pl.BlockSpec((_E, _TB), lambda i: (0, i)),
        out_shape=jax.ShapeDtypeStruct((_E, n), jnp.float32),
        compiler_params=pltpu.CompilerParams(
            dimension_semantics=("arbitrary",),
        ),
    )(x, W)


def _sc_top2_body(tpw, lg_hbm, i1_hbm, i2_hbm, w0_hbm, w1_hbm,
                  lg_v, i1_v, i2_v, w0_v, w1_v):
    wid = lax.axis_index("s") * _NC + lax.axis_index("c")
    base = wid * tpw
    pltpu.sync_copy(lg_hbm.at[:, pl.ds(base, tpw)], lg_v)

    @plsc.parallel_loop(0, tpw // _L, unroll=4)
    def group(g):
        tok = g * _L
        neg = jnp.full((_L,), -jnp.inf, jnp.float32)
        zero = jnp.zeros((_L,), jnp.int32)
        bestv, secondv = neg, neg
        besti, secondi = zero, zero
        for e in range(_E):
            col = jnp.full((_L,), e, jnp.int32)
            v = lg_v[e, pl.ds(tok, _L)]
            gt1 = v > bestv
            gt2 = v > secondv
            sv = jnp.maximum(secondv, v)
            si = jnp.where(gt2, col, secondi)
            secondv = jnp.where(gt1, bestv, sv)
            secondi = jnp.where(gt1, besti, si)
            bestv = jnp.maximum(bestv, v)
            besti = jnp.where(gt1, col, besti)
        ex = jnp.exp(secondv - bestv)
        w0 = 1.0 / (1.0 + ex)
        i1_v[pl.ds(tok, _L)] = besti
        i2_v[pl.ds(tok, _L)] = secondi
        w0_v[pl.ds(tok, _L)] = w0
        w1_v[pl.ds(tok, _L)] = 1.0 - w0
    pltpu.sync_copy(i1_v, i1_hbm.at[pl.ds(base, tpw)])
    pltpu.sync_copy(i2_v, i2_hbm.at[pl.ds(base, tpw)])
    pltpu.sync_copy(w0_v, w0_hbm.at[pl.ds(base, tpw)])
    pltpu.sync_copy(w1_v, w1_hbm.at[pl.ds(base, tpw)])


def _sc_top2(logits_t):
    n = logits_t.shape[1]
    tpw = n // _NW
    mesh = plsc.VectorSubcoreMesh(core_axis_name="c", subcore_axis_name="s")
    out_i = jax.ShapeDtypeStruct((n,), jnp.int32)
    out_f = jax.ShapeDtypeStruct((n,), jnp.float32)
    fn = pl.kernel(
        functools.partial(_sc_top2_body, tpw),
        out_type=[out_i, out_i, out_f, out_f],
        mesh=mesh,
        scratch_types=[
            pltpu.VMEM((_E, tpw), jnp.float32),
            pltpu.VMEM((tpw,), jnp.int32),
            pltpu.VMEM((tpw,), jnp.int32),
            pltpu.VMEM((tpw,), jnp.float32),
            pltpu.VMEM((tpw,), jnp.float32),
        ],
    )
    return fn(logits_t)


def kernel(x, W):
    logits_t = _gate_logits_t(x, W)
    i1, i2, w0, w1 = _sc_top2(logits_t)
    topi = jnp.stack([i1, i2], axis=1)
    weights = jnp.stack([w0, w1], axis=1)
    return (topi, weights)


# speedup vs baseline: 1.1911x; 1.0204x over previous
import functools

import jax
import jax.numpy as jnp
from jax import lax
from jax.experimental import pallas as pl
from jax.experimental.pallas import tpu as pltpu
from jax.experimental.pallas import tpu_sc as plsc

_E = 64
_TB = 1024
_NC = 2
_NS = 16
_NW = _NC * _NS
_L = 16


def _matmul_body(x_ref, w_ref, lg_ref):
    lg_ref[...] = jax.lax.dot_general(
        w_ref[...], x_ref[...], (((1,), (1,)), ((), ())),
        preferred_element_type=jnp.float32)


def _gate_logits_t(x, W):
    n, d = x.shape
    return pl.pallas_call(
        _matmul_body,
        grid=(n // _TB,),
        in_specs=[
            pl.BlockSpec((_TB, d), lambda i: (i, 0)),
            pl.BlockSpec((_E, d), lambda i: (0, 0)),
        ],
        out_specs=pl.BlockSpec((_E, _TB), lambda i: (0, i)),
        out_shape=jax.ShapeDtypeStruct((_E, n), jnp.float32),
        compiler_params=pltpu.CompilerParams(
            dimension_semantics=("arbitrary",),
        ),
    )(x, W)


def _sc_top2_body(tpw, lg_hbm, out_hbm, lg_v, o_v):
    wid = lax.axis_index("s") * _NC + lax.axis_index("c")
    base = wid * tpw
    pltpu.sync_copy(lg_hbm.at[:, pl.ds(base, tpw)], lg_v)

    @plsc.parallel_loop(0, tpw // _L, unroll=4)
    def group(g):
        tok = g * _L
        neg = jnp.full((_L,), -jnp.inf, jnp.float32)
        zero = jnp.zeros((_L,), jnp.int32)
        bestv, secondv = neg, neg
        besti, secondi = zero, zero
        for e in range(_E):
            col = jnp.full((_L,), e, jnp.int32)
            v = lg_v[e, pl.ds(tok, _L)]
            gt1 = v > bestv
            gt2 = v > secondv
            sv = jnp.maximum(secondv, v)
            si = jnp.where(gt2, col, secondi)
            secondv = jnp.where(gt1, bestv, sv)
            secondi = jnp.where(gt1, besti, si)
            bestv = jnp.maximum(bestv, v)
            besti = jnp.where(gt1, col, besti)
        ex = jnp.exp(secondv - bestv)
        w0 = 1.0 / (1.0 + ex)
        o_v[0, pl.ds(tok, _L)] = besti.astype(jnp.float32)
        o_v[1, pl.ds(tok, _L)] = secondi.astype(jnp.float32)
        o_v[2, pl.ds(tok, _L)] = w0
        o_v[3, pl.ds(tok, _L)] = 1.0 - w0
    pltpu.sync_copy(o_v, out_hbm.at[:, pl.ds(base, tpw)])


def _sc_top2(logits_t):
    n = logits_t.shape[1]
    tpw = n // _NW
    mesh = plsc.VectorSubcoreMesh(core_axis_name="c", subcore_axis_name="s")
    fn = pl.kernel(
        functools.partial(_sc_top2_body, tpw),
        out_type=jax.ShapeDtypeStruct((4, n), jnp.float32),
        mesh=mesh,
        scratch_types=[
            pltpu.VMEM((_E, tpw), jnp.float32),
            pltpu.VMEM((4, tpw), jnp.float32),
        ],
    )
    return fn(logits_t)


def kernel(x, W):
    logits_t = _gate_logits_t(x, W)
    out = _sc_top2(logits_t)
    topi = out[0:2].T.astype(jnp.int32)
    weights = out[2:4].T
    return (topi, weights)
